# baseline (speedup 1.0000x reference)
import functools

import jax
import jax.numpy as jnp
from jax import lax
from jax.experimental import pallas as pl
from jax.experimental.pallas import tpu as pltpu
from jax.experimental.pallas import tpu_sc as plsc

N = 10000
E = 320000
IN_FEATS = 128
HIDDEN = 128
MSG = 64
OUT_FEATS = 128
AUG = 80

NC = 2
NS = 16
NW = NC * NS
EPW = E // NW
CHUNK = 80
NCHUNK = EPW // CHUNK
NBUF = 8
DEPTH = NBUF // 2
NPAD = 10240
ROWS_PER_TILE = NPAD // NS

ROW_BLOCK = 1000
GRID = N // ROW_BLOCK


def _mlp_body(x_ref, w1t_ref, b1_ref, w2t_ref, b2_ref, m_ref):
    h1 = jnp.dot(x_ref[...], w1t_ref[...], preferred_element_type=jnp.float32)
    h1 = jnp.maximum(h1 + b1_ref[...], 0.0)
    m = jnp.dot(h1, w2t_ref[...], preferred_element_type=jnp.float32)
    m = jnp.maximum(m + b2_ref[...], 0.0)
    one = jnp.ones((ROW_BLOCK, 1), jnp.float32)
    pad = jnp.zeros((ROW_BLOCK, AUG - MSG - 1), jnp.float32)
    m_ref[...] = jnp.concatenate([m, one, pad], axis=1)


def _node_mlp(x, w1t, b1, w2t, b2):
    return pl.pallas_call(
        _mlp_body,
        grid=(GRID,),
        in_specs=[
            pl.BlockSpec((ROW_BLOCK, IN_FEATS), lambda i: (i, 0)),
            pl.BlockSpec((IN_FEATS, HIDDEN), lambda i: (0, 0)),
            pl.BlockSpec((1, HIDDEN), lambda i: (0, 0)),
            pl.BlockSpec((HIDDEN, MSG), lambda i: (0, 0)),
            pl.BlockSpec((1, MSG), lambda i: (0, 0)),
        ],
        out_specs=pl.BlockSpec((ROW_BLOCK, AUG), lambda i: (i, 0)),
        out_shape=jax.ShapeDtypeStruct((N, AUG), jnp.float32),
    )(x, w1t, b1, w2t, b2)


def _sc_agg_body(m_hbm, src_hbm, dst_hbm, acc_out,
                 src_v, dst_v, rows_v, acc_sh, sem_g, sem_s):
    c = lax.axis_index("c")
    s = lax.axis_index("s")
    wid = s * NC + c

    z16 = jnp.zeros((16,), jnp.float32)

    def _fill_zeros(i, carry):
        for k in range(AUG // 16):
            rows_v[0, i, pl.ds(16 * k, 16)] = z16
        return carry

    lax.fori_loop(0, CHUNK, _fill_zeros, 0)

    base_row = s * ROWS_PER_TILE
    for r in range(ROWS_PER_TILE // CHUNK):
        pltpu.sync_copy(rows_v.at[0], acc_sh.at[pl.ds(base_row + r * CHUNK, CHUNK)])
    plsc.subcore_barrier()

    pltpu.sync_copy(src_hbm.at[wid], src_v)
    pltpu.sync_copy(dst_hbm.at[wid], dst_v)

    for p in range(DEPTH):
        pltpu.async_copy(m_hbm.at[src_v.at[p]], rows_v.at[p], sem_g)

    def _chunk(j, carry):
        b = lax.rem(j, NBUF)
        pltpu.make_async_copy(m_hbm.at[src_v.at[j]], rows_v.at[b], sem_g).wait()
        pltpu.async_copy(rows_v.at[b], acc_sh.at[dst_v.at[j]], sem_s, add=True)

        @pl.when(j >= DEPTH)
        def _wait_old_scatter():
            bo = lax.rem(j - DEPTH, NBUF)
            pltpu.make_async_copy(rows_v.at[bo], acc_sh.at[dst_v.at[j - DEPTH]],
                                  sem_s).wait()

        @pl.when(j + DEPTH < NCHUNK)
        def _start_next_gather():
            bn = lax.rem(j + DEPTH, NBUF)
            pltpu.async_copy(m_hbm.at[src_v.at[j + DEPTH]], rows_v.at[bn], sem_g)

        return carry

    lax.fori_loop(0, NCHUNK, _chunk, 0)
    for p in range(DEPTH):
        jj = NCHUNK - DEPTH + p
        pltpu.make_async_copy(rows_v.at[jj % NBUF],
                              acc_sh.at[dst_v.at[jj]], sem_s).wait()
    plsc.subcore_barrier()

    pltpu.sync_copy(acc_sh.at[pl.ds(base_row, ROWS_PER_TILE)],
                    acc_out.at[c, pl.ds(base_row, ROWS_PER_TILE)])


_sc_agg = functools.partial(
    pl.kernel,
    out_type=jax.ShapeDtypeStruct((NC, NPAD, AUG), jnp.float32),
    mesh=plsc.VectorSubcoreMesh(core_axis_name="c", subcore_axis_name="s"),
    scratch_types=[
        pltpu.VMEM((NCHUNK, CHUNK), jnp.int32),
        pltpu.VMEM((NCHUNK, CHUNK), jnp.int32),
        pltpu.VMEM((NBUF, CHUNK, AUG), jnp.float32),
        pltpu.VMEM_SHARED((NPAD, AUG), jnp.float32),
        pltpu.SemaphoreType.DMA,
        pltpu.SemaphoreType.DMA,
    ],
    compiler_params=pltpu.CompilerParams(use_tc_tiling_on_sc=False),
)(_sc_agg_body)


def _gru_body(x_ref, z_ref, acc_ref, wixt_ref, wiyt_ref, whht_ref,
              bih_ref, bhh_ref, out_ref):
    acc = acc_ref[...]
    msum = acc[0] + acc[1]
    y = msum[:, :MSG] / jnp.maximum(msum[:, MSG:MSG + 1], 1.0)

    gi = jnp.dot(x_ref[...], wixt_ref[...], preferred_element_type=jnp.float32)
    gi = gi + jnp.dot(y, wiyt_ref[...], preferred_element_type=jnp.float32)
    gi = gi + bih_ref[...]
    gh = jnp.dot(z_ref[...], whht_ref[...], preferred_element_type=jnp.float32)
    gh = gh + bhh_ref[...]

    r = jax.nn.sigmoid(gi[:, :OUT_FEATS] + gh[:, :OUT_FEATS])
    u = jax.nn.sigmoid(gi[:, OUT_FEATS:2 * OUT_FEATS] + gh[:, OUT_FEATS:2 * OUT_FEATS])
    n = jnp.tanh(gi[:, 2 * OUT_FEATS:] + r * gh[:, 2 * OUT_FEATS:])
    out_ref[...] = (1.0 - u) * n + u * z_ref[...]


def _gru_update(x, z, acc, wixt, wiyt, whht, bih, bhh):
    return pl.pallas_call(
        _gru_body,
        grid=(GRID,),
        in_specs=[
            pl.BlockSpec((ROW_BLOCK, IN_FEATS), lambda i: (i, 0)),
            pl.BlockSpec((ROW_BLOCK, OUT_FEATS), lambda i: (i, 0)),
            pl.BlockSpec((NC, ROW_BLOCK, AUG), lambda i: (0, i, 0)),
            pl.BlockSpec((IN_FEATS, 3 * OUT_FEATS), lambda i: (0, 0)),
            pl.BlockSpec((MSG, 3 * OUT_FEATS), lambda i: (0, 0)),
            pl.BlockSpec((OUT_FEATS, 3 * OUT_FEATS), lambda i: (0, 0)),
            pl.BlockSpec((1, 3 * OUT_FEATS), lambda i: (0, 0)),
            pl.BlockSpec((1, 3 * OUT_FEATS), lambda i: (0, 0)),
        ],
        out_specs=pl.BlockSpec((ROW_BLOCK, OUT_FEATS), lambda i: (i, 0)),
        out_shape=jax.ShapeDtypeStruct((N, OUT_FEATS), jnp.float32),
    )(x, z, acc, wixt, wiyt, whht, bih, bhh)


def kernel(x, z, edge_index, W1, b1, W2, b2, Wih, Whh, bih, bhh):
    src = edge_index[0].reshape(NW, NCHUNK, CHUNK)
    dst = edge_index[1].reshape(NW, NCHUNK, CHUNK)

    m = _node_mlp(x, W1.T, b1.reshape(1, HIDDEN), W2.T, b2.reshape(1, MSG))
    acc = _sc_agg(m, src, dst)
    h_out = _gru_update(
        x, z, acc,
        Wih[:, :IN_FEATS].T, Wih[:, IN_FEATS:].T, Whh.T,
        bih.reshape(1, 3 * OUT_FEATS), bhh.reshape(1, 3 * OUT_FEATS),
    )
    return (h_out, h_out)


# speedup vs baseline: 1.0845x; 1.0845x over previous
import functools

import jax
import jax.numpy as jnp
from jax import lax
from jax.experimental import pallas as pl
from jax.experimental.pallas import tpu as pltpu
from jax.experimental.pallas import tpu_sc as plsc

N = 10000
E = 320000
IN_FEATS = 128
HIDDEN = 128
MSG = 64
OUT_FEATS = 128
AUG = 80

NC = 2
NS = 16
NW = NC * NS
EPW = E // NW
CHUNK = 80
NCHUNK = EPW // CHUNK
NBUF = 8
DEPTH = NBUF // 2
NPAD = 10240
ROWS_PER_TILE = NPAD // NS

ROW_BLOCK = 1000
GRID = N // ROW_BLOCK


def _mlp_body(x_ref, w1t_ref, b1_ref, w2t_ref, b2_ref, m_ref):
    h1 = jnp.dot(x_ref[...], w1t_ref[...], preferred_element_type=jnp.float32)
    h1 = jnp.maximum(h1 + b1_ref[...], 0.0)
    m = jnp.dot(h1, w2t_ref[...], preferred_element_type=jnp.float32)
    m = jnp.maximum(m + b2_ref[...], 0.0)
    one = jnp.ones((ROW_BLOCK, 1), jnp.float32)
    pad = jnp.zeros((ROW_BLOCK, AUG - MSG - 1), jnp.float32)
    m_ref[...] = jnp.concatenate([m, one, pad], axis=1)


def _node_mlp(x, w1t, b1, w2t, b2):
    return pl.pallas_call(
        _mlp_body,
        grid=(GRID,),
        in_specs=[
            pl.BlockSpec((ROW_BLOCK, IN_FEATS), lambda i: (i, 0)),
            pl.BlockSpec((IN_FEATS, HIDDEN), lambda i: (0, 0)),
            pl.BlockSpec((1, HIDDEN), lambda i: (0, 0)),
            pl.BlockSpec((HIDDEN, MSG), lambda i: (0, 0)),
            pl.BlockSpec((1, MSG), lambda i: (0, 0)),
        ],
        out_specs=pl.BlockSpec((ROW_BLOCK, AUG), lambda i: (i, 0)),
        out_shape=jax.ShapeDtypeStruct((N, AUG), jnp.float32),
    )(x, w1t, b1, w2t, b2)


def _sc_agg_body(m_hbm, src_hbm, dst_hbm, acc_out,
                 src_v, dst_v, rows_v, acc_sh, sem_g, sem_s):
    c = lax.axis_index("c")
    s = lax.axis_index("s")
    wid = s * NC + c

    z16 = jnp.zeros((16,), jnp.float32)

    def _fill_zeros(i, carry):
        for k in range(AUG // 16):
            rows_v[0, i, pl.ds(16 * k, 16)] = z16
        return carry

    lax.fori_loop(0, CHUNK, _fill_zeros, 0)

    base_row = s * ROWS_PER_TILE
    for r in range(ROWS_PER_TILE // CHUNK):
        pltpu.sync_copy(rows_v.at[0], acc_sh.at[pl.ds(base_row + r * CHUNK, CHUNK)])
    plsc.subcore_barrier()

    pltpu.sync_copy(src_hbm.at[wid], src_v)
    pltpu.sync_copy(dst_hbm.at[wid], dst_v)

    for p in range(DEPTH):
        pltpu.async_copy(m_hbm.at[src_v.at[p]], rows_v.at[p], sem_g)

    def _chunk(j, carry):
        b = lax.rem(j, NBUF)
        pltpu.make_async_copy(m_hbm.at[src_v.at[j]], rows_v.at[b], sem_g).wait()
        pltpu.async_copy(rows_v.at[b], acc_sh.at[dst_v.at[j]], sem_s, add=True)

        @pl.when(j >= DEPTH)
        def _wait_old_scatter():
            bo = lax.rem(j - DEPTH, NBUF)
            pltpu.make_async_copy(rows_v.at[bo], acc_sh.at[dst_v.at[j - DEPTH]],
                                  sem_s).wait()

        @pl.when(j + DEPTH < NCHUNK)
        def _start_next_gather():
            bn = lax.rem(j + DEPTH, NBUF)
            pltpu.async_copy(m_hbm.at[src_v.at[j + DEPTH]], rows_v.at[bn], sem_g)

        return carry

    lax.fori_loop(0, NCHUNK, _chunk, 0)
    for p in range(DEPTH):
        jj = NCHUNK - DEPTH + p
        pltpu.make_async_copy(rows_v.at[jj % NBUF],
                              acc_sh.at[dst_v.at[jj]], sem_s).wait()
    plsc.subcore_barrier()

    pltpu.sync_copy(acc_sh.at[pl.ds(base_row, ROWS_PER_TILE)],
                    acc_out.at[c, pl.ds(base_row, ROWS_PER_TILE), pl.ds(0, AUG)])


_sc_agg = functools.partial(
    pl.kernel,
    out_type=jax.ShapeDtypeStruct((NC, NPAD, 128), jnp.float32),
    mesh=plsc.VectorSubcoreMesh(core_axis_name="c", subcore_axis_name="s"),
    scratch_types=[
        pltpu.VMEM((NCHUNK, CHUNK), jnp.int32),
        pltpu.VMEM((NCHUNK, CHUNK), jnp.int32),
        pltpu.VMEM((NBUF, CHUNK, AUG), jnp.float32),
        pltpu.VMEM_SHARED((NPAD, AUG), jnp.float32),
        pltpu.SemaphoreType.DMA,
        pltpu.SemaphoreType.DMA,
    ],
    compiler_params=pltpu.CompilerParams(use_tc_tiling_on_sc=False),
)(_sc_agg_body)


def _gru_body(x_ref, z_ref, acc_ref, wixt_ref, wiyt_ref, whht_ref,
              bih_ref, bhh_ref, out_ref):
    acc = acc_ref[...]
    msum = acc[0] + acc[1]
    y = msum[:, :MSG] / jnp.maximum(msum[:, MSG:MSG + 1], 1.0)

    gi = jnp.dot(x_ref[...], wixt_ref[...], preferred_element_type=jnp.float32)
    gi = gi + jnp.dot(y, wiyt_ref[...], preferred_element_type=jnp.float32)
    gi = gi + bih_ref[...]
    gh = jnp.dot(z_ref[...], whht_ref[...], preferred_element_type=jnp.float32)
    gh = gh + bhh_ref[...]

    r = jax.nn.sigmoid(gi[:, :OUT_FEATS] + gh[:, :OUT_FEATS])
    u = jax.nn.sigmoid(gi[:, OUT_FEATS:2 * OUT_FEATS] + gh[:, OUT_FEATS:2 * OUT_FEATS])
    n = jnp.tanh(gi[:, 2 * OUT_FEATS:] + r * gh[:, 2 * OUT_FEATS:])
    out_ref[...] = (1.0 - u) * n + u * z_ref[...]


def _gru_update(x, z, acc, wixt, wiyt, whht, bih, bhh):
    return pl.pallas_call(
        _gru_body,
        grid=(GRID,),
        in_specs=[
            pl.BlockSpec((ROW_BLOCK, IN_FEATS), lambda i: (i, 0)),
            pl.BlockSpec((ROW_BLOCK, OUT_FEATS), lambda i: (i, 0)),
            pl.BlockSpec((NC, ROW_BLOCK, 128), lambda i: (0, i, 0)),
            pl.BlockSpec((IN_FEATS, 3 * OUT_FEATS), lambda i: (0, 0)),
            pl.BlockSpec((MSG, 3 * OUT_FEATS), lambda i: (0, 0)),
            pl.BlockSpec((OUT_FEATS, 3 * OUT_FEATS), lambda i: (0, 0)),
            pl.BlockSpec((1, 3 * OUT_FEATS), lambda i: (0, 0)),
            pl.BlockSpec((1, 3 * OUT_FEATS), lambda i: (0, 0)),
        ],
        out_specs=pl.BlockSpec((ROW_BLOCK, OUT_FEATS), lambda i: (i, 0)),
        out_shape=jax.ShapeDtypeStruct((N, OUT_FEATS), jnp.float32),
    )(x, z, acc, wixt, wiyt, whht, bih, bhh)


def kernel(x, z, edge_index, W1, b1, W2, b2, Wih, Whh, bih, bhh):
    src = edge_index[0].reshape(NW, NCHUNK, CHUNK)
    dst = edge_index[1].reshape(NW, NCHUNK, CHUNK)

    m = _node_mlp(x, W1.T, b1.reshape(1, HIDDEN), W2.T, b2.reshape(1, MSG))
    acc = _sc_agg(m, src, dst)
    h_out = _gru_update(
        x, z, acc,
        Wih[:, :IN_FEATS].T, Wih[:, IN_FEATS:].T, Whh.T,
        bih.reshape(1, 3 * OUT_FEATS), bhh.reshape(1, 3 * OUT_FEATS),
    )
    return (h_out, h_out)


# speedup vs baseline: 1.0986x; 1.0131x over previous
import functools

import jax
import jax.numpy as jnp
from jax import lax
from jax.experimental import pallas as pl
from jax.experimental.pallas import tpu as pltpu
from jax.experimental.pallas import tpu_sc as plsc

N = 10000
E = 320000
IN_FEATS = 128
HIDDEN = 128
MSG = 64
OUT_FEATS = 128
AUG = 80

NC = 2
NS = 16
NW = NC * NS
EPW = E // NW
CHUNK = 80
NCHUNK = EPW // CHUNK
NBUF = 8
DEPTH = NBUF // 2
NPAD = 10240
ROWS_PER_TILE = NPAD // NS

ROW_BLOCK = 1000
GRID = N // ROW_BLOCK


def _mlp_body(x_ref, w1t_ref, b1_ref, w2t_ref, b2_ref, m_ref):
    xb = x_ref[...].astype(jnp.bfloat16)
    h1 = jnp.dot(xb, w1t_ref[...], preferred_element_type=jnp.float32)
    h1 = jnp.maximum(h1 + b1_ref[...], 0.0)
    m = jnp.dot(h1.astype(jnp.bfloat16), w2t_ref[...],
                preferred_element_type=jnp.float32)
    m = jnp.maximum(m + b2_ref[...], 0.0)
    one = jnp.ones((ROW_BLOCK, 1), jnp.float32)
    pad = jnp.zeros((ROW_BLOCK, AUG - MSG - 1), jnp.float32)
    m_ref[...] = jnp.concatenate([m, one, pad], axis=1)


def _node_mlp(x, w1t, b1, w2t, b2):
    return pl.pallas_call(
        _mlp_body,
        grid=(GRID,),
        in_specs=[
            pl.BlockSpec((ROW_BLOCK, IN_FEATS), lambda i: (i, 0)),
            pl.BlockSpec((IN_FEATS, HIDDEN), lambda i: (0, 0)),
            pl.BlockSpec((1, HIDDEN), lambda i: (0, 0)),
            pl.BlockSpec((HIDDEN, MSG), lambda i: (0, 0)),
            pl.BlockSpec((1, MSG), lambda i: (0, 0)),
        ],
        out_specs=pl.BlockSpec((ROW_BLOCK, AUG), lambda i: (i, 0)),
        out_shape=jax.ShapeDtypeStruct((N, AUG), jnp.float32),
    )(x, w1t, b1, w2t, b2)


def _sc_agg_body(m_hbm, src_hbm, dst_hbm, acc_out,
                 src_v, dst_v, rows_v, acc_sh, sem_g, sem_s):
    c = lax.axis_index("c")
    s = lax.axis_index("s")
    wid = s * NC + c

    z16 = jnp.zeros((16,), jnp.float32)

    def _fill_zeros(i, carry):
        for k in range(AUG // 16):
            rows_v[0, i, pl.ds(16 * k, 16)] = z16
        return carry

    lax.fori_loop(0, CHUNK, _fill_zeros, 0)

    base_row = s * ROWS_PER_TILE
    for r in range(ROWS_PER_TILE // CHUNK):
        pltpu.sync_copy(rows_v.at[0], acc_sh.at[pl.ds(base_row + r * CHUNK, CHUNK)])
    plsc.subcore_barrier()

    pltpu.sync_copy(src_hbm.at[wid], src_v)
    pltpu.sync_copy(dst_hbm.at[wid], dst_v)

    for p in range(DEPTH):
        pltpu.async_copy(m_hbm.at[src_v.at[p]], rows_v.at[p], sem_g)

    def _chunk(j, carry):
        b = lax.rem(j, NBUF)
        pltpu.make_async_copy(m_hbm.at[src_v.at[j]], rows_v.at[b], sem_g).wait()
        pltpu.async_copy(rows_v.at[b], acc_sh.at[dst_v.at[j]], sem_s, add=True)

        @pl.when(j >= DEPTH)
        def _wait_old_scatter():
            bo = lax.rem(j - DEPTH, NBUF)
            pltpu.make_async_copy(rows_v.at[bo], acc_sh.at[dst_v.at[j - DEPTH]],
                                  sem_s).wait()

        @pl.when(j + DEPTH < NCHUNK)
        def _start_next_gather():
            bn = lax.rem(j + DEPTH, NBUF)
            pltpu.async_copy(m_hbm.at[src_v.at[j + DEPTH]], rows_v.at[bn], sem_g)

        return carry

    lax.fori_loop(0, NCHUNK, _chunk, 0)
    for p in range(DEPTH):
        jj = NCHUNK - DEPTH + p
        pltpu.make_async_copy(rows_v.at[jj % NBUF],
                              acc_sh.at[dst_v.at[jj]], sem_s).wait()
    plsc.subcore_barrier()

    pltpu.sync_copy(acc_sh.at[pl.ds(base_row, ROWS_PER_TILE)],
                    acc_out.at[c, pl.ds(base_row, ROWS_PER_TILE), pl.ds(0, AUG)])


_sc_agg = functools.partial(
    pl.kernel,
    out_type=jax.ShapeDtypeStruct((NC, NPAD, 128), jnp.float32),
    mesh=plsc.VectorSubcoreMesh(core_axis_name="c", subcore_axis_name="s"),
    scratch_types=[
        pltpu.VMEM((NCHUNK, CHUNK), jnp.int32),
        pltpu.VMEM((NCHUNK, CHUNK), jnp.int32),
        pltpu.VMEM((NBUF, CHUNK, AUG), jnp.float32),
        pltpu.VMEM_SHARED((NPAD, AUG), jnp.float32),
        pltpu.SemaphoreType.DMA,
        pltpu.SemaphoreType.DMA,
    ],
    compiler_params=pltpu.CompilerParams(use_tc_tiling_on_sc=False),
)(_sc_agg_body)


def _gru_body(x_ref, z_ref, acc_ref, wixt_ref, wiyt_ref, whht_ref,
              bih_ref, bhh_ref, out_ref, out2_ref):
    acc = acc_ref[...]
    msum = acc[0] + acc[1]
    y = msum[:, :MSG] / jnp.maximum(msum[:, MSG:MSG + 1], 1.0)

    gi = jnp.dot(x_ref[...], wixt_ref[...], preferred_element_type=jnp.float32)
    gi = gi + jnp.dot(y, wiyt_ref[...], preferred_element_type=jnp.float32)
    gi = gi + bih_ref[...]
    gh = jnp.dot(z_ref[...], whht_ref[...], preferred_element_type=jnp.float32)
    gh = gh + bhh_ref[...]

    r = jax.nn.sigmoid(gi[:, :OUT_FEATS] + gh[:, :OUT_FEATS])
    u = jax.nn.sigmoid(gi[:, OUT_FEATS:2 * OUT_FEATS] + gh[:, OUT_FEATS:2 * OUT_FEATS])
    n = jnp.tanh(gi[:, 2 * OUT_FEATS:] + r * gh[:, 2 * OUT_FEATS:])
    h = (1.0 - u) * n + u * z_ref[...]
    out_ref[...] = h
    out2_ref[...] = h


def _gru_update(x, z, acc, wixt, wiyt, whht, bih, bhh):
    return pl.pallas_call(
        _gru_body,
        grid=(GRID,),
        in_specs=[
            pl.BlockSpec((ROW_BLOCK, IN_FEATS), lambda i: (i, 0)),
            pl.BlockSpec((ROW_BLOCK, OUT_FEATS), lambda i: (i, 0)),
            pl.BlockSpec((NC, ROW_BLOCK, 128), lambda i: (0, i, 0)),
            pl.BlockSpec((IN_FEATS, 3 * OUT_FEATS), lambda i: (0, 0)),
            pl.BlockSpec((MSG, 3 * OUT_FEATS), lambda i: (0, 0)),
            pl.BlockSpec((OUT_FEATS, 3 * OUT_FEATS), lambda i: (0, 0)),
            pl.BlockSpec((1, 3 * OUT_FEATS), lambda i: (0, 0)),
            pl.BlockSpec((1, 3 * OUT_FEATS), lambda i: (0, 0)),
        ],
        out_specs=[pl.BlockSpec((ROW_BLOCK, OUT_FEATS), lambda i: (i, 0)),
                   pl.BlockSpec((ROW_BLOCK, OUT_FEATS), lambda i: (i, 0))],
        out_shape=[jax.ShapeDtypeStruct((N, OUT_FEATS), jnp.float32),
                   jax.ShapeDtypeStruct((N, OUT_FEATS), jnp.float32)],
    )(x, z, acc, wixt, wiyt, whht, bih, bhh)


def kernel(x, z, edge_index, W1, b1, W2, b2, Wih, Whh, bih, bhh):
    src = edge_index[0].reshape(NW, NCHUNK, CHUNK)
    dst = edge_index[1].reshape(NW, NCHUNK, CHUNK)

    m = _node_mlp(x, W1.T.astype(jnp.bfloat16), b1.reshape(1, HIDDEN),
                  W2.T.astype(jnp.bfloat16), b2.reshape(1, MSG))
    acc = _sc_agg(m, src, dst)
    h_out, h_out2 = _gru_update(
        x, z, acc,
        Wih[:, :IN_FEATS].T, Wih[:, IN_FEATS:].T, Whh.T,
        bih.reshape(1, 3 * OUT_FEATS), bhh.reshape(1, 3 * OUT_FEATS),
    )
    return (h_out, h_out2)
